# Initial kernel scaffold; baseline (speedup 1.0000x reference)
import functools

import jax
import jax.numpy as jnp
from jax import lax
from jax.experimental import pallas as pl
from jax.experimental.pallas import tpu as pltpu
from jax.experimental.pallas import tpu_sc as plsc

B = 8
N = 2048
C = 3
K = 30
KP = 32
RB = 256

NEG = float("-inf")


def _topk_body(xt_ref, x_ref, idx_ref):
    xtb = xt_ref[0]
    xb = x_ref[0]
    xj0 = xb[0:1, :]
    xj1 = xb[1:2, :]
    xj2 = xb[2:3, :]
    xx_j = xj0 * xj0 + xj1 * xj1 + xj2 * xj2
    xi0 = xtb[:, 0:1]
    xi1 = xtb[:, 1:2]
    xi2 = xtb[:, 2:3]
    xx_i = xi0 * xi0 + xi1 * xi1 + xi2 * xi2
    inner = -2.0 * (xi0 * xj0 + xi1 * xj1 + xi2 * xj2)
    d = -xx_i - inner - xx_j

    iota = lax.broadcasted_iota(jnp.int32, (RB, N), 1)
    cols = []
    for _ in range(K):
        m = jnp.max(d, axis=1, keepdims=True)
        cand = jnp.where(d == m, iota, N)
        j = jnp.min(cand, axis=1, keepdims=True)
        cols.append(j)
        d = jnp.where(iota == j, NEG, d)
    cols.append(jnp.zeros((RB, 1), jnp.int32))
    cols.append(jnp.zeros((RB, 1), jnp.int32))
    idx_ref[0] = jnp.concatenate(cols, axis=1)


def _topk_indices(xt, x):
    return pl.pallas_call(
        _topk_body,
        grid=(B, N // RB),
        in_specs=[
            pl.BlockSpec((1, RB, C), lambda b, r: (b, r, 0)),
            pl.BlockSpec((1, C, N), lambda b, r: (b, 0, 0)),
        ],
        out_specs=pl.BlockSpec((1, RB, KP), lambda b, r: (b, r, 0)),
        out_shape=jax.ShapeDtypeStruct((B, N, KP), jnp.int32),
        compiler_params=pltpu.CompilerParams(
            dimension_semantics=("parallel", "parallel"),
        ),
    )(xt, x)


NW = 32
ROWS_W = B * N // NW
CH = 64
W_PER_B = N // ROWS_W


def _sc_feature_kernel(xt_hbm, idx_hbm, out_hbm, tab, idxv, outv, sem):
    del sem
    wid = lax.axis_index("s") * 2 + lax.axis_index("c")
    b = wid // W_PER_B
    rbase = (wid % W_PER_B) * ROWS_W
    pltpu.sync_copy(xt_hbm.at[b], tab)

    def chunk(ci, carry):
        base = rbase + ci * CH
        pltpu.sync_copy(idx_hbm.at[b, pl.ds(base, CH)], idxv)

        def row(r, carry2):
            cx = tab[base + r, 0]
            cy = tab[base + r, 1]
            cz = tab[base + r, 2]
            cxv = jnp.broadcast_to(cx, (16,))
            cyv = jnp.broadcast_to(cy, (16,))
            czv = jnp.broadcast_to(cz, (16,))
            for h in range(KP // 16):
                nb = idxv[r, pl.ds(16 * h, 16)]
                zero = jnp.zeros((16,), jnp.int32)
                gx = plsc.load_gather(tab, [nb, zero])
                gy = plsc.load_gather(tab, [nb, zero + 1])
                gz = plsc.load_gather(tab, [nb, zero + 2])
                rx = gx - cxv
                ry = gy - cyv
                rz = gz - czv
                sq = rx * rx + ry * ry + rz * rz
                bits = lax.bitcast_convert_type(sq, jnp.int32)
                y = lax.bitcast_convert_type(
                    jnp.full((16,), 0x5F3759DF, jnp.int32)
                    - lax.shift_right_arithmetic(bits, 1),
                    jnp.float32,
                )
                for _ in range(3):
                    y = y * (1.5 - 0.5 * sq * y * y)
                rn = sq * y + 1e-08
                sl = pl.ds(16 * h, 16)
                outv[r, 0, sl] = rx
                outv[r, 1, sl] = ry
                outv[r, 2, sl] = rz
                outv[r, 3, sl] = rx / rn
                outv[r, 4, sl] = ry / rn
                outv[r, 5, sl] = rz / rn
                outv[r, 6, sl] = cxv
                outv[r, 7, sl] = cyv
                outv[r, 8, sl] = czv
            return carry2

        lax.fori_loop(0, CH, row, 0)
        pltpu.sync_copy(outv, out_hbm.at[b, pl.ds(base, CH)])
        return carry

    lax.fori_loop(0, ROWS_W // CH, chunk, 0)


def _sc_features(xt, idx):
    mesh = plsc.VectorSubcoreMesh(core_axis_name="c", subcore_axis_name="s")
    kfn = functools.partial(
        pl.kernel,
        mesh=mesh,
        out_type=jax.ShapeDtypeStruct((B, N, 9, KP), jnp.float32),
        scratch_types=[
            pltpu.VMEM((N, C), jnp.float32),
            pltpu.VMEM((CH, KP), jnp.int32),
            pltpu.VMEM((CH, 9, KP), jnp.float32),
            pltpu.SemaphoreType.DMA,
        ],
    )(_sc_feature_kernel)
    return kfn(xt, idx)


def kernel(x):
    xt = jnp.transpose(x, (0, 2, 1))
    idx = _topk_indices(xt, x)
    feats = _sc_features(xt, idx)
    f2 = jnp.transpose(feats[:, :, :, :30], (0, 2, 1, 3))
    f1 = jnp.transpose(feats[:, :, :, :20], (0, 2, 1, 3))
    f0 = jnp.transpose(
        jnp.concatenate([feats[:, :, 0:3, :5], feats[:, :, 6:9, :5]], axis=2),
        (0, 2, 1, 3),
    )
    return (f0, f1, f2)


# speedup vs baseline: 15.3449x; 15.3449x over previous
import functools

import jax
import jax.numpy as jnp
from jax import lax
from jax.experimental import pallas as pl
from jax.experimental.pallas import tpu as pltpu
from jax.experimental.pallas import tpu_sc as plsc

B = 8
N = 2048
C = 3
K = 30
KP = 32
RB = 256

NEG = float("-inf")


def _topk_body(xt_ref, x_ref, idx_ref):
    xtb = xt_ref[0]
    xb = x_ref[0]
    xj0 = xb[0:1, :]
    xj1 = xb[1:2, :]
    xj2 = xb[2:3, :]
    xx_j = xj0 * xj0 + xj1 * xj1 + xj2 * xj2
    xi0 = xtb[:, 0:1]
    xi1 = xtb[:, 1:2]
    xi2 = xtb[:, 2:3]
    xx_i = xi0 * xi0 + xi1 * xi1 + xi2 * xi2
    inner = -2.0 * lax.dot_general(
        xtb.astype(jnp.bfloat16),
        xb.astype(jnp.bfloat16),
        (((1,), (0,)), ((), ())),
        preferred_element_type=jnp.float32,
    )
    d = -xx_i - inner - xx_j

    iota = lax.broadcasted_iota(jnp.int32, (RB, N), 1)
    cols = []
    for _ in range(K):
        m = jnp.max(d, axis=1, keepdims=True)
        cand = jnp.where(d == m, iota, N)
        j = jnp.min(cand, axis=1, keepdims=True)
        cols.append(j)
        d = jnp.where(iota == j, NEG, d)
    cols.append(jnp.zeros((RB, 1), jnp.int32))
    cols.append(jnp.zeros((RB, 1), jnp.int32))
    idx_ref[0] = jnp.concatenate(cols, axis=1)


def _topk_indices(xt, x):
    return pl.pallas_call(
        _topk_body,
        grid=(B, N // RB),
        in_specs=[
            pl.BlockSpec((1, RB, C), lambda b, r: (b, r, 0)),
            pl.BlockSpec((1, C, N), lambda b, r: (b, 0, 0)),
        ],
        out_specs=pl.BlockSpec((1, RB, KP), lambda b, r: (b, r, 0)),
        out_shape=jax.ShapeDtypeStruct((B, N, KP), jnp.int32),
        compiler_params=pltpu.CompilerParams(
            dimension_semantics=("parallel", "parallel"),
        ),
    )(xt, x)


NW = 32
ROWS_W = B * N // NW
CH = 64
W_PER_B = N // ROWS_W


def _sc_feature_kernel(x_hbm, idx_hbm, out_hbm, tab0, tab1, tab2, idxv, outv, sem):
    del sem
    wid = lax.axis_index("s") * 2 + lax.axis_index("c")
    b = wid // W_PER_B
    rbase = (wid % W_PER_B) * ROWS_W
    pltpu.sync_copy(x_hbm.at[pl.ds((b * 3 + 0) * N, N)], tab0)
    pltpu.sync_copy(x_hbm.at[pl.ds((b * 3 + 1) * N, N)], tab1)
    pltpu.sync_copy(x_hbm.at[pl.ds((b * 3 + 2) * N, N)], tab2)

    def chunk(ci, carry):
        base = rbase + ci * CH
        gbase = b * N + base
        pltpu.sync_copy(idx_hbm.at[pl.ds(gbase, CH)], idxv)

        def row(r, carry2):
            ctr = jnp.broadcast_to(base + r, (16,))
            cxv = plsc.load_gather(tab0, [ctr])
            cyv = plsc.load_gather(tab1, [ctr])
            czv = plsc.load_gather(tab2, [ctr])
            for h in range(KP // 16):
                nb = idxv[r, pl.ds(16 * h, 16)]
                gx = plsc.load_gather(tab0, [nb])
                gy = plsc.load_gather(tab1, [nb])
                gz = plsc.load_gather(tab2, [nb])
                rx = gx - cxv
                ry = gy - cyv
                rz = gz - czv
                sq = rx * rx + ry * ry + rz * rz
                bits = lax.bitcast_convert_type(sq, jnp.int32)
                y = lax.bitcast_convert_type(
                    jnp.full((16,), 0x5F3759DF, jnp.int32)
                    - lax.shift_right_arithmetic(bits, 1),
                    jnp.float32,
                )
                for _ in range(3):
                    y = y * (1.5 - 0.5 * sq * y * y)
                rn = sq * y + 1e-08
                sl = pl.ds(16 * h, 16)
                outv[r, 0, sl] = rx
                outv[r, 1, sl] = ry
                outv[r, 2, sl] = rz
                outv[r, 3, sl] = rx / rn
                outv[r, 4, sl] = ry / rn
                outv[r, 5, sl] = rz / rn
                outv[r, 6, sl] = cxv
                outv[r, 7, sl] = cyv
                outv[r, 8, sl] = czv
            return carry2

        lax.fori_loop(0, CH, row, 0)
        pltpu.sync_copy(outv, out_hbm.at[pl.ds(gbase, CH)])
        return carry

    lax.fori_loop(0, ROWS_W // CH, chunk, 0)


def _sc_features(x, idx):
    mesh = plsc.VectorSubcoreMesh(core_axis_name="c", subcore_axis_name="s")
    kfn = functools.partial(
        pl.kernel,
        mesh=mesh,
        out_type=jax.ShapeDtypeStruct((B * N, 9, KP), jnp.float32),
        scratch_types=[
            pltpu.VMEM((N,), jnp.float32),
            pltpu.VMEM((N,), jnp.float32),
            pltpu.VMEM((N,), jnp.float32),
            pltpu.VMEM((CH, KP), jnp.int32),
            pltpu.VMEM((CH, 9, KP), jnp.float32),
            pltpu.SemaphoreType.DMA,
        ],
        compiler_params=pltpu.CompilerParams(
            needs_layout_passes=False, use_tc_tiling_on_sc=False
        ),
    )(_sc_feature_kernel)
    return kfn(x.reshape(B * 3 * N), idx.reshape(B * N, KP)).reshape(B, N, 9, KP)


def kernel(x):
    xt = jnp.transpose(x, (0, 2, 1))
    idx = _topk_indices(xt, x)
    feats = _sc_features(x, idx)
    f2 = jnp.transpose(feats[:, :, :, :30], (0, 2, 1, 3))
    f1 = jnp.transpose(feats[:, :, :, :20], (0, 2, 1, 3))
    f0 = jnp.transpose(
        jnp.concatenate([feats[:, :, 0:3, :5], feats[:, :, 6:9, :5]], axis=2),
        (0, 2, 1, 3),
    )
    return (f0, f1, f2)


# speedup vs baseline: 20.5171x; 1.3371x over previous
import functools

import jax
import jax.numpy as jnp
from jax import lax
from jax.experimental import pallas as pl
from jax.experimental.pallas import tpu as pltpu
from jax.experimental.pallas import tpu_sc as plsc

B = 8
N = 2048
C = 3
K = 30
KP = 32
RB = 256

NEG = float("-inf")


def _topk_body(xt_ref, x_ref, idx_ref):
    xtb = xt_ref[0]
    xb = x_ref[0]
    xj0 = xb[0:1, :]
    xj1 = xb[1:2, :]
    xj2 = xb[2:3, :]
    xx_j = xj0 * xj0 + xj1 * xj1 + xj2 * xj2
    xi0 = xtb[:, 0:1]
    xi1 = xtb[:, 1:2]
    xi2 = xtb[:, 2:3]
    xx_i = xi0 * xi0 + xi1 * xi1 + xi2 * xi2
    inner = -2.0 * lax.dot_general(
        xtb.astype(jnp.bfloat16),
        xb.astype(jnp.bfloat16),
        (((1,), (0,)), ((), ())),
        preferred_element_type=jnp.float32,
    )
    d = -xx_i - inner - xx_j

    NC = 8
    W = N // NC
    iota_p = lax.broadcasted_iota(jnp.int32, (RB, W), 1)
    vals = [d[:, c * W : (c + 1) * W] for c in range(NC)]
    cid = [jnp.full((RB, W), c, jnp.int32) for c in range(NC)]

    network = [
        (0, 1), (2, 3), (4, 5), (6, 7),
        (0, 2), (1, 3), (1, 2),
        (4, 6), (5, 7), (5, 6),
        (0, 4), (1, 5), (2, 6), (3, 7),
        (2, 4), (3, 5),
        (1, 2), (3, 4), (5, 6),
    ]
    for i, k in network:
        vi, vk = vals[i], vals[k]
        ci, ck = cid[i], cid[k]
        sw = (vk > vi) | ((vk == vi) & (ck < ci))
        vals[i] = jnp.where(sw, vk, vi)
        vals[k] = jnp.where(sw, vi, vk)
        cid[i] = jnp.where(sw, ck, ci)
        cid[k] = jnp.where(sw, ci, ck)

    packed = cid[0]
    for lvl in range(1, NC):
        packed = packed | (cid[lvl] << (4 * lvl))

    BIG = jnp.int32(1 << 30)
    cols = []
    for _ in range(K):
        top = vals[0]
        gid0 = ((packed & 7) << 8) | iota_p
        m = jnp.max(top, axis=1, keepdims=True)
        cand = jnp.where(top == m, gid0, BIG)
        g = jnp.min(cand, axis=1, keepdims=True)
        cols.append(g)
        pm = iota_p == (g & (W - 1))
        for lvl in range(NC - 1):
            vals[lvl] = jnp.where(pm, vals[lvl + 1], vals[lvl])
        vals[NC - 1] = jnp.where(pm, NEG, vals[NC - 1])
        packed = jnp.where(pm, lax.shift_right_logical(packed, 4), packed)
    cols.append(jnp.zeros((RB, 1), jnp.int32))
    cols.append(jnp.zeros((RB, 1), jnp.int32))
    idx_ref[0] = jnp.concatenate(cols, axis=1)


def _topk_indices(xt, x):
    return pl.pallas_call(
        _topk_body,
        grid=(B, N // RB),
        in_specs=[
            pl.BlockSpec((1, RB, C), lambda b, r: (b, r, 0)),
            pl.BlockSpec((1, C, N), lambda b, r: (b, 0, 0)),
        ],
        out_specs=pl.BlockSpec((1, RB, KP), lambda b, r: (b, r, 0)),
        out_shape=jax.ShapeDtypeStruct((B, N, KP), jnp.int32),
        compiler_params=pltpu.CompilerParams(
            dimension_semantics=("parallel", "parallel"),
        ),
    )(xt, x)


NW = 32
ROWS_W = B * N // NW
CH = 64
W_PER_B = N // ROWS_W


def _sc_feature_kernel(x_hbm, idx_hbm, out_hbm, tab0, tab1, tab2, idxv, outv, sem):
    del sem
    wid = lax.axis_index("s") * 2 + lax.axis_index("c")
    b = wid // W_PER_B
    rbase = (wid % W_PER_B) * ROWS_W
    pltpu.sync_copy(x_hbm.at[pl.ds((b * 3 + 0) * N, N)], tab0)
    pltpu.sync_copy(x_hbm.at[pl.ds((b * 3 + 1) * N, N)], tab1)
    pltpu.sync_copy(x_hbm.at[pl.ds((b * 3 + 2) * N, N)], tab2)

    def chunk(ci, carry):
        base = rbase + ci * CH
        gbase = b * N + base
        pltpu.sync_copy(idx_hbm.at[pl.ds(gbase, CH)], idxv)

        def row(r, carry2):
            ctr = jnp.broadcast_to(base + r, (16,))
            cxv = plsc.load_gather(tab0, [ctr])
            cyv = plsc.load_gather(tab1, [ctr])
            czv = plsc.load_gather(tab2, [ctr])
            for h in range(KP // 16):
                nb = idxv[r, pl.ds(16 * h, 16)]
                gx = plsc.load_gather(tab0, [nb])
                gy = plsc.load_gather(tab1, [nb])
                gz = plsc.load_gather(tab2, [nb])
                rx = gx - cxv
                ry = gy - cyv
                rz = gz - czv
                sq = rx * rx + ry * ry + rz * rz
                bits = lax.bitcast_convert_type(sq, jnp.int32)
                y = lax.bitcast_convert_type(
                    jnp.full((16,), 0x5F3759DF, jnp.int32)
                    - lax.shift_right_arithmetic(bits, 1),
                    jnp.float32,
                )
                for _ in range(3):
                    y = y * (1.5 - 0.5 * sq * y * y)
                rn = sq * y + 1e-08
                sl = pl.ds(16 * h, 16)
                outv[r, 0, sl] = rx
                outv[r, 1, sl] = ry
                outv[r, 2, sl] = rz
                outv[r, 3, sl] = rx / rn
                outv[r, 4, sl] = ry / rn
                outv[r, 5, sl] = rz / rn
                outv[r, 6, sl] = cxv
                outv[r, 7, sl] = cyv
                outv[r, 8, sl] = czv
            return carry2

        lax.fori_loop(0, CH, row, 0)
        pltpu.sync_copy(outv, out_hbm.at[pl.ds(gbase, CH)])
        return carry

    lax.fori_loop(0, ROWS_W // CH, chunk, 0)


def _sc_features(x, idx):
    mesh = plsc.VectorSubcoreMesh(core_axis_name="c", subcore_axis_name="s")
    kfn = functools.partial(
        pl.kernel,
        mesh=mesh,
        out_type=jax.ShapeDtypeStruct((B * N, 9, KP), jnp.float32),
        scratch_types=[
            pltpu.VMEM((N,), jnp.float32),
            pltpu.VMEM((N,), jnp.float32),
            pltpu.VMEM((N,), jnp.float32),
            pltpu.VMEM((CH, KP), jnp.int32),
            pltpu.VMEM((CH, 9, KP), jnp.float32),
            pltpu.SemaphoreType.DMA,
        ],
        compiler_params=pltpu.CompilerParams(
            needs_layout_passes=False, use_tc_tiling_on_sc=False
        ),
    )(_sc_feature_kernel)
    return kfn(x.reshape(B * 3 * N), idx.reshape(B * N, KP)).reshape(B, N, 9, KP)


def kernel(x):
    xt = jnp.transpose(x, (0, 2, 1))
    idx = _topk_indices(xt, x)
    feats = _sc_features(x, idx)
    f2 = jnp.transpose(feats[:, :, :, :30], (0, 2, 1, 3))
    f1 = jnp.transpose(feats[:, :, :, :20], (0, 2, 1, 3))
    f0 = jnp.transpose(
        jnp.concatenate([feats[:, :, 0:3, :5], feats[:, :, 6:9, :5]], axis=2),
        (0, 2, 1, 3),
    )
    return (f0, f1, f2)


# speedup vs baseline: 22.5057x; 1.0969x over previous
import functools

import jax
import jax.numpy as jnp
from jax import lax
from jax.experimental import pallas as pl
from jax.experimental.pallas import tpu as pltpu
from jax.experimental.pallas import tpu_sc as plsc

B = 8
N = 2048
C = 3
K = 30
KP = 32
RB = 256

NEG = float("-inf")


def _topk_body(xt_ref, x_ref, idx_ref):
    xtb = xt_ref[0]
    xb = x_ref[0]
    xj0 = xb[0:1, :]
    xj1 = xb[1:2, :]
    xj2 = xb[2:3, :]
    xx_j = xj0 * xj0 + xj1 * xj1 + xj2 * xj2
    xi0 = xtb[:, 0:1]
    xi1 = xtb[:, 1:2]
    xi2 = xtb[:, 2:3]
    xx_i = xi0 * xi0 + xi1 * xi1 + xi2 * xi2
    inner = -2.0 * lax.dot_general(
        xtb.astype(jnp.bfloat16),
        xb.astype(jnp.bfloat16),
        (((1,), (0,)), ((), ())),
        preferred_element_type=jnp.float32,
    )
    d = -xx_i - inner - xx_j

    NC = 8
    W = N // NC
    iota_p = lax.broadcasted_iota(jnp.int32, (RB, W), 1)
    vals = [d[:, c * W : (c + 1) * W] for c in range(NC)]
    cid = [jnp.full((RB, W), c, jnp.int32) for c in range(NC)]

    network = [
        (0, 1), (2, 3), (4, 5), (6, 7),
        (0, 2), (1, 3), (1, 2),
        (4, 6), (5, 7), (5, 6),
        (0, 4), (1, 5), (2, 6), (3, 7),
        (2, 4), (3, 5),
        (1, 2), (3, 4), (5, 6),
    ]
    for i, k in network:
        vi, vk = vals[i], vals[k]
        ci, ck = cid[i], cid[k]
        sw = (vk > vi) | ((vk == vi) & (ck < ci))
        vals[i] = jnp.where(sw, vk, vi)
        vals[k] = jnp.where(sw, vi, vk)
        cid[i] = jnp.where(sw, ck, ci)
        cid[k] = jnp.where(sw, ci, ck)

    packed = cid[0]
    for lvl in range(1, NC):
        packed = packed | (cid[lvl] << (4 * lvl))

    BIG = jnp.float32(1 << 30)
    iota_f = iota_p.astype(jnp.float32)
    cols = []
    for _ in range(K):
        top = vals[0]
        gid0 = (packed & 7).astype(jnp.float32) * float(W) + iota_f
        m = jnp.max(top, axis=1, keepdims=True)
        cand = jnp.where(top == m, gid0, BIG)
        gf = jnp.min(cand, axis=1, keepdims=True)
        cols.append(gf)
        pm = cand == gf
        for lvl in range(NC - 1):
            vals[lvl] = jnp.where(pm, vals[lvl + 1], vals[lvl])
        vals[NC - 1] = jnp.where(pm, NEG, vals[NC - 1])
        packed = jnp.where(pm, lax.shift_right_logical(packed, 4), packed)
    cols.append(jnp.zeros((RB, 1), jnp.float32))
    cols.append(jnp.zeros((RB, 1), jnp.float32))
    idx_ref[0] = jnp.concatenate(cols, axis=1).astype(jnp.int32)


def _topk_indices(xt, x):
    return pl.pallas_call(
        _topk_body,
        grid=(B, N // RB),
        in_specs=[
            pl.BlockSpec((1, RB, C), lambda b, r: (b, r, 0)),
            pl.BlockSpec((1, C, N), lambda b, r: (b, 0, 0)),
        ],
        out_specs=pl.BlockSpec((1, RB, KP), lambda b, r: (b, r, 0)),
        out_shape=jax.ShapeDtypeStruct((B, N, KP), jnp.int32),
        compiler_params=pltpu.CompilerParams(
            dimension_semantics=("parallel", "parallel"),
        ),
    )(xt, x)


NW = 32
ROWS_W = B * N // NW
CH = 64
W_PER_B = N // ROWS_W


def _sc_feature_kernel(x_hbm, idx_hbm, out_hbm, tab0, tab1, tab2, idxv, outv, sem):
    del sem
    wid = lax.axis_index("s") * 2 + lax.axis_index("c")
    b = wid // W_PER_B
    rbase = (wid % W_PER_B) * ROWS_W
    pltpu.sync_copy(x_hbm.at[pl.ds((b * 3 + 0) * N, N)], tab0)
    pltpu.sync_copy(x_hbm.at[pl.ds((b * 3 + 1) * N, N)], tab1)
    pltpu.sync_copy(x_hbm.at[pl.ds((b * 3 + 2) * N, N)], tab2)

    def chunk(ci, carry):
        base = rbase + ci * CH
        gbase = b * N + base
        pltpu.sync_copy(idx_hbm.at[pl.ds(gbase, CH)], idxv)

        def row(r, carry2):
            ctr = jnp.broadcast_to(base + r, (16,))
            cxv = plsc.load_gather(tab0, [ctr])
            cyv = plsc.load_gather(tab1, [ctr])
            czv = plsc.load_gather(tab2, [ctr])
            for h in range(KP // 16):
                nb = idxv[r, pl.ds(16 * h, 16)]
                gx = plsc.load_gather(tab0, [nb])
                gy = plsc.load_gather(tab1, [nb])
                gz = plsc.load_gather(tab2, [nb])
                rx = gx - cxv
                ry = gy - cyv
                rz = gz - czv
                sq = rx * rx + ry * ry + rz * rz
                bits = lax.bitcast_convert_type(sq, jnp.int32)
                y = lax.bitcast_convert_type(
                    jnp.full((16,), 0x5F3759DF, jnp.int32)
                    - lax.shift_right_arithmetic(bits, 1),
                    jnp.float32,
                )
                for _ in range(3):
                    y = y * (1.5 - 0.5 * sq * y * y)
                rn = sq * y + 1e-08
                sl = pl.ds(16 * h, 16)
                outv[r, 0, sl] = rx
                outv[r, 1, sl] = ry
                outv[r, 2, sl] = rz
                outv[r, 3, sl] = rx / rn
                outv[r, 4, sl] = ry / rn
                outv[r, 5, sl] = rz / rn
                outv[r, 6, sl] = cxv
                outv[r, 7, sl] = cyv
                outv[r, 8, sl] = czv
            return carry2

        lax.fori_loop(0, CH, row, 0)
        pltpu.sync_copy(outv, out_hbm.at[pl.ds(gbase, CH)])
        return carry

    lax.fori_loop(0, ROWS_W // CH, chunk, 0)


def _sc_features(x, idx):
    mesh = plsc.VectorSubcoreMesh(core_axis_name="c", subcore_axis_name="s")
    kfn = functools.partial(
        pl.kernel,
        mesh=mesh,
        out_type=jax.ShapeDtypeStruct((B * N, 9, KP), jnp.float32),
        scratch_types=[
            pltpu.VMEM((N,), jnp.float32),
            pltpu.VMEM((N,), jnp.float32),
            pltpu.VMEM((N,), jnp.float32),
            pltpu.VMEM((CH, KP), jnp.int32),
            pltpu.VMEM((CH, 9, KP), jnp.float32),
            pltpu.SemaphoreType.DMA,
        ],
        compiler_params=pltpu.CompilerParams(
            needs_layout_passes=False, use_tc_tiling_on_sc=False
        ),
    )(_sc_feature_kernel)
    return kfn(x.reshape(B * 3 * N), idx.reshape(B * N, KP)).reshape(B, N, 9, KP)


def kernel(x):
    xt = jnp.transpose(x, (0, 2, 1))
    idx = _topk_indices(xt, x)
    feats = _sc_features(x, idx)
    f2 = jnp.transpose(feats[:, :, :, :30], (0, 2, 1, 3))
    f1 = jnp.transpose(feats[:, :, :, :20], (0, 2, 1, 3))
    f0 = jnp.transpose(
        jnp.concatenate([feats[:, :, 0:3, :5], feats[:, :, 6:9, :5]], axis=2),
        (0, 2, 1, 3),
    )
    return (f0, f1, f2)


# speedup vs baseline: 24.9808x; 1.1100x over previous
import functools

import jax
import jax.numpy as jnp
from jax import lax
from jax.experimental import pallas as pl
from jax.experimental.pallas import tpu as pltpu
from jax.experimental.pallas import tpu_sc as plsc

B = 8
N = 2048
C = 3
K = 30
KP = 32
RB = 256

NEG = float("-inf")


def _topk_body(xt_ref, x_ref, idx_ref):
    xtb = xt_ref[0]
    xb = x_ref[0]
    xj0 = xb[0:1, :]
    xj1 = xb[1:2, :]
    xj2 = xb[2:3, :]
    xx_j = xj0 * xj0 + xj1 * xj1 + xj2 * xj2
    xi0 = xtb[:, 0:1]
    xi1 = xtb[:, 1:2]
    xi2 = xtb[:, 2:3]
    xx_i = xi0 * xi0 + xi1 * xi1 + xi2 * xi2
    inner = -2.0 * lax.dot_general(
        xtb.astype(jnp.bfloat16),
        xb.astype(jnp.bfloat16),
        (((1,), (0,)), ((), ())),
        preferred_element_type=jnp.float32,
    )
    d = -xx_i - inner - xx_j

    NC = 8
    W = N // NC
    iota_p = lax.broadcasted_iota(jnp.int32, (RB, W), 1)
    vals = [d[:, c * W : (c + 1) * W] for c in range(NC)]
    cid = [jnp.full((RB, W), c, jnp.int32) for c in range(NC)]

    network = [
        (0, 1), (2, 3), (4, 5), (6, 7),
        (0, 2), (1, 3), (1, 2),
        (4, 6), (5, 7), (5, 6),
        (0, 4), (1, 5), (2, 6), (3, 7),
        (2, 4), (3, 5),
        (1, 2), (3, 4), (5, 6),
    ]
    for i, k in network:
        vi, vk = vals[i], vals[k]
        ci, ck = cid[i], cid[k]
        sw = (vk > vi) | ((vk == vi) & (ck < ci))
        vals[i] = jnp.where(sw, vk, vi)
        vals[k] = jnp.where(sw, vi, vk)
        cid[i] = jnp.where(sw, ck, ci)
        cid[k] = jnp.where(sw, ci, ck)

    packed = cid[0]
    for lvl in range(1, NC):
        packed = packed | (cid[lvl] << (4 * lvl))

    BIG = jnp.float32(1 << 30)
    iota_f = iota_p.astype(jnp.float32)
    cols = []
    for _ in range(K):
        top = vals[0]
        gid0 = (packed & 7).astype(jnp.float32) * float(W) + iota_f
        m = jnp.max(top, axis=1, keepdims=True)
        cand = jnp.where(top == m, gid0, BIG)
        gf = jnp.min(cand, axis=1, keepdims=True)
        cols.append(gf)
        pm = cand == gf
        for lvl in range(NC - 1):
            vals[lvl] = jnp.where(pm, vals[lvl + 1], vals[lvl])
        vals[NC - 1] = jnp.where(pm, NEG, vals[NC - 1])
        packed = jnp.where(pm, lax.shift_right_logical(packed, 4), packed)
    cols.append(jnp.zeros((RB, 1), jnp.float32))
    cols.append(jnp.zeros((RB, 1), jnp.float32))
    idx_ref[0] = jnp.concatenate(cols, axis=1).astype(jnp.int32)


def _topk_indices(xt, x):
    return pl.pallas_call(
        _topk_body,
        grid=(B, N // RB),
        in_specs=[
            pl.BlockSpec((1, RB, C), lambda b, r: (b, r, 0)),
            pl.BlockSpec((1, C, N), lambda b, r: (b, 0, 0)),
        ],
        out_specs=pl.BlockSpec((1, RB, KP), lambda b, r: (b, r, 0)),
        out_shape=jax.ShapeDtypeStruct((B, N, KP), jnp.int32),
        compiler_params=pltpu.CompilerParams(
            dimension_semantics=("parallel", "parallel"),
        ),
    )(xt, x)


NW = 32
ROWS_W = B * N // NW
CH = 64
W_PER_B = N // ROWS_W


def _sc_feature_kernel(x_hbm, idx_hbm, out_hbm, tab0, tab1, tab2, idxv, outv, sem):
    del sem
    wid = lax.axis_index("s") * 2 + lax.axis_index("c")
    b = wid // W_PER_B
    rbase = (wid % W_PER_B) * ROWS_W
    pltpu.sync_copy(x_hbm.at[pl.ds((b * 3 + 0) * N, N)], tab0)
    pltpu.sync_copy(x_hbm.at[pl.ds((b * 3 + 1) * N, N)], tab1)
    pltpu.sync_copy(x_hbm.at[pl.ds((b * 3 + 2) * N, N)], tab2)

    def chunk(ci, carry):
        base = rbase + ci * CH
        gbase = b * N + base
        pltpu.sync_copy(idx_hbm.at[pl.ds(gbase, CH)], idxv)

        def row(r, carry2):
            ctr = jnp.broadcast_to(base + r, (16,))
            cxv = plsc.load_gather(tab0, [ctr])
            cyv = plsc.load_gather(tab1, [ctr])
            czv = plsc.load_gather(tab2, [ctr])
            for h in range(KP // 16):
                nb = idxv[r, pl.ds(16 * h, 16)]
                gx = plsc.load_gather(tab0, [nb])
                gy = plsc.load_gather(tab1, [nb])
                gz = plsc.load_gather(tab2, [nb])
                rx = gx - cxv
                ry = gy - cyv
                rz = gz - czv
                sq = rx * rx + ry * ry + rz * rz
                bits = lax.bitcast_convert_type(sq, jnp.int32)
                y = lax.bitcast_convert_type(
                    jnp.full((16,), 0x5F3759DF, jnp.int32)
                    - lax.shift_right_arithmetic(bits, 1),
                    jnp.float32,
                )
                for _ in range(3):
                    y = y * (1.5 - 0.5 * sq * y * y)
                rn = sq * y + 1e-08
                sl = pl.ds(16 * h, 16)
                outv[r, 0, sl] = rx
                outv[r, 1, sl] = ry
                outv[r, 2, sl] = rz
                outv[r, 3, sl] = rx / rn
                outv[r, 4, sl] = ry / rn
                outv[r, 5, sl] = rz / rn
                outv[r, 6, sl] = cxv
                outv[r, 7, sl] = cyv
                outv[r, 8, sl] = czv
            return carry2

        lax.fori_loop(0, CH, row, 0)
        pltpu.sync_copy(outv, out_hbm.at[pl.ds(gbase, CH)])
        return carry

    lax.fori_loop(0, ROWS_W // CH, chunk, 0)


def _sc_features(x, idx):
    mesh = plsc.VectorSubcoreMesh(core_axis_name="c", subcore_axis_name="s")
    kfn = functools.partial(
        pl.kernel,
        mesh=mesh,
        out_type=jax.ShapeDtypeStruct((B * N, 9, KP), jnp.float32),
        scratch_types=[
            pltpu.VMEM((N,), jnp.float32),
            pltpu.VMEM((N,), jnp.float32),
            pltpu.VMEM((N,), jnp.float32),
            pltpu.VMEM((CH, KP), jnp.int32),
            pltpu.VMEM((CH, 9, KP), jnp.float32),
            pltpu.SemaphoreType.DMA,
        ],
        compiler_params=pltpu.CompilerParams(
            needs_layout_passes=False, use_tc_tiling_on_sc=False
        ),
    )(_sc_feature_kernel)
    return kfn(x.reshape(B * 3 * N), idx.reshape(B * N, KP)).reshape(B, N, 9, KP)


def kernel(x):
    xt = jnp.transpose(x, (0, 2, 1))
    idx = _topk_indices(xt, x)
    feats = _sc_features(x, idx)
    ft = jnp.transpose(feats, (0, 2, 1, 3))
    f2 = ft[:, :, :, :30]
    f1 = ft[:, :, :, :20]
    f0 = jnp.concatenate((ft[:, 0:3, :, :5], ft[:, 6:9, :, :5]), axis=1)
    return (f0, f1, f2)
